# baseline (speedup 1.0000x reference)
import functools

import jax
import jax.numpy as jnp
from jax import lax
from jax.experimental import pallas as pl
from jax.experimental.pallas import tpu as pltpu

H = 64
NG = 64
MAX_DEG = 64
NB_LAYER = 2
N = 50000
E = 800000
EP = 802816
ERB = 4096
NRB = 2000

_f32 = jnp.float32


def _full(shape):
    return pl.BlockSpec(shape, lambda i: tuple(0 for _ in shape))


def _rows(rb, c):
    return pl.BlockSpec((rb, c), lambda i: (i, 0))


def _mlp3_body(x_ref, w0_ref, b0_ref, w1_ref, b1_ref, w2_ref, b2_ref, o_ref):
    x = x_ref[...]
    if x.shape[1] == 1:
        h = jnp.maximum(x * w0_ref[...] + b0_ref[...], 0.0)
    else:
        h = jnp.maximum(jnp.dot(x, w0_ref[...], preferred_element_type=_f32) + b0_ref[...], 0.0)
    h = jnp.maximum(jnp.dot(h, w1_ref[...], preferred_element_type=_f32) + b1_ref[...], 0.0)
    o_ref[...] = jnp.dot(h, w2_ref[...], preferred_element_type=_f32) + b2_ref[...]


def _mlp3(p, x, rb):
    r, cin = x.shape
    cout = p["W"][2].shape[1]
    ws = [p["W"][0], p["b"][0].reshape(1, -1), p["W"][1], p["b"][1].reshape(1, -1),
          p["W"][2], p["b"][2].reshape(1, -1)]
    return pl.pallas_call(
        _mlp3_body,
        grid=(r // rb,),
        in_specs=[_rows(rb, cin)] + [_full(w.shape) for w in ws],
        out_specs=_rows(rb, cout),
        out_shape=jax.ShapeDtypeStruct((r, cout), _f32),
    )(x, *ws)


def _qkvs_body(x_ref, wq, bq, wk, bk, wv, bv, ws, bs, q_ref, k_ref, v_ref, s_ref):
    x = x_ref[...]
    q_ref[...] = jnp.dot(x, wq[...], preferred_element_type=_f32) + bq[...]
    k_ref[...] = jnp.dot(x, wk[...], preferred_element_type=_f32) + bk[...]
    v_ref[...] = jnp.dot(x, wv[...], preferred_element_type=_f32) + bv[...]
    s_ref[...] = jnp.dot(x, ws[...], preferred_element_type=_f32) + bs[...]


def _qkvs(p, x):
    ws = [p["Wq"], p["bq"].reshape(1, -1), p["Wk"], p["bk"].reshape(1, -1),
          p["Wv"], p["bv"].reshape(1, -1), p["Wskip"], p["bskip"].reshape(1, -1)]
    out = jax.ShapeDtypeStruct((N, H), _f32)
    return pl.pallas_call(
        _qkvs_body,
        grid=(N // NRB,),
        in_specs=[_rows(NRB, 2 * H)] + [_full(w.shape) for w in ws],
        out_specs=[_rows(NRB, H)] * 4,
        out_shape=[out] * 4,
    )(x, *ws)


def _eproj_body(e_ref, w_ref, b_ref, o_ref):
    o_ref[...] = jnp.dot(e_ref[...], w_ref[...], preferred_element_type=_f32) + b_ref[...]


def _eproj(p, ee):
    ws = [p["We"], p["be"].reshape(1, -1)]
    return pl.pallas_call(
        _eproj_body,
        grid=(EP // ERB,),
        in_specs=[_rows(ERB, H)] + [_full(w.shape) for w in ws],
        out_specs=_rows(ERB, H),
        out_shape=jax.ShapeDtypeStruct((EP, H), _f32),
    )(ee, *ws)


def _nodeenc_body(x2_ref, b_ref, demb_ref, tval_ref,
                  tw0, tb0, tw1, tb1, tw2, tb2,
                  nw0, nb0, nw1, nb1, nw2, nb2,
                  te_ref, ne_ref):
    deg = jnp.clip(x2_ref[...], 0, MAX_DEG)
    oh_d = (deg == lax.broadcasted_iota(jnp.int32, (deg.shape[0], MAX_DEG + 1), 1)).astype(_f32)
    demb = jnp.dot(oh_d, demb_ref[...], preferred_element_type=_f32)
    oh_b = (b_ref[...] == lax.broadcasted_iota(jnp.int32, (deg.shape[0], NG), 1)).astype(_f32)
    tn = jnp.dot(oh_b, tval_ref[...], preferred_element_type=_f32)
    h = jnp.maximum(tn * tw0[...] + tb0[...], 0.0)
    h = jnp.maximum(jnp.dot(h, tw1[...], preferred_element_type=_f32) + tb1[...], 0.0)
    te_ref[...] = jnp.dot(h, tw2[...], preferred_element_type=_f32) + tb2[...]
    g = jnp.maximum(jnp.dot(demb, nw0[...], preferred_element_type=_f32) + nb0[...], 0.0)
    g = jnp.maximum(jnp.dot(g, nw1[...], preferred_element_type=_f32) + nb1[...], 0.0)
    ne_ref[...] = jnp.dot(g, nw2[...], preferred_element_type=_f32) + nb2[...]


def _nodeenc(params, x2, batch1, t_value):
    tp, np_ = params["time"], params["node"]
    ws = [params["deg_emb"], t_value.reshape(NG, 1),
          tp["W"][0], tp["b"][0].reshape(1, -1), tp["W"][1], tp["b"][1].reshape(1, -1),
          tp["W"][2], tp["b"][2].reshape(1, -1),
          np_["W"][0], np_["b"][0].reshape(1, -1), np_["W"][1], np_["b"][1].reshape(1, -1),
          np_["W"][2], np_["b"][2].reshape(1, -1)]
    out = jax.ShapeDtypeStruct((N, H), _f32)
    return pl.pallas_call(
        _nodeenc_body,
        grid=(N // NRB,),
        in_specs=[_rows(NRB, 1), _rows(NRB, 1)] + [_full(w.shape) for w in ws],
        out_specs=[_rows(NRB, H)] * 2,
        out_shape=[out] * 2,
    )(x2, batch1, *ws)


def _alpha_body(qd_ref, ks_ref, ee_ref, a_ref, gm_ref):
    i = pl.program_id(0)
    s = jnp.sum(qd_ref[...] * (ks_ref[...] + ee_ref[...]), axis=1, keepdims=True) * 0.125
    rid = i * ERB + lax.broadcasted_iota(jnp.int32, (ERB, 1), 0)
    s = jnp.where(rid < E, s, -1e30)
    a_ref[...] = s
    bm = jnp.max(s, axis=(0, 1), keepdims=True)

    @pl.when(i == 0)
    def _():
        gm_ref[...] = jnp.full((1, 1), -1e30, _f32)

    gm_ref[...] = jnp.maximum(gm_ref[...], bm)


def _alpha(qd, ks, ee):
    return pl.pallas_call(
        _alpha_body,
        grid=(EP // ERB,),
        in_specs=[_rows(ERB, H)] * 3,
        out_specs=[_rows(ERB, 1), pl.BlockSpec((1, 1), lambda i: (0, 0))],
        out_shape=[jax.ShapeDtypeStruct((EP, 1), _f32), jax.ShapeDtypeStruct((1, 1), _f32)],
    )(qd, ks, ee)


def _msg_body(a_ref, vs_ref, ee_ref, o_ref):
    o_ref[...] = a_ref[...] * (vs_ref[...] + ee_ref[...])


def _msg(a, vs, ee):
    return pl.pallas_call(
        _msg_body,
        grid=(EP // ERB,),
        in_specs=[_rows(ERB, 1), _rows(ERB, H), _rows(ERB, H)],
        out_specs=_rows(ERB, H),
        out_shape=jax.ShapeDtypeStruct((EP, H), _f32),
    )(a, vs, ee)


def _gn1_body(sc_ref, sk_ref, b_ref, s_ref, c_ref):
    i = pl.program_id(0)
    x = sc_ref[...] + sk_ref[...]
    oh = (b_ref[...] == lax.broadcasted_iota(jnp.int32, (x.shape[0], NG), 1)).astype(_f32)

    @pl.when(i == 0)
    def _():
        s_ref[...] = jnp.zeros_like(s_ref)
        c_ref[...] = jnp.zeros_like(c_ref)

    s_ref[...] += lax.dot_general(oh, x, (((0,), (0,)), ((), ())), preferred_element_type=_f32)
    c_ref[...] += jnp.sum(oh, axis=0, keepdims=True)


def _gn2_body(sc_ref, sk_ref, b_ref, s_ref, c_ref, al_ref, xc_ref, v_ref):
    i = pl.program_id(0)
    x = sc_ref[...] + sk_ref[...]
    oh = (b_ref[...] == lax.broadcasted_iota(jnp.int32, (x.shape[0], NG), 1)).astype(_f32)
    cnt = c_ref[...].reshape(NG, 1) + 1e-6
    mean = s_ref[...] / cnt
    xc = x - al_ref[...] * jnp.dot(oh, mean, preferred_element_type=_f32)
    xc_ref[...] = xc

    @pl.when(i == 0)
    def _():
        v_ref[...] = jnp.zeros_like(v_ref)

    v_ref[...] += lax.dot_general(oh, xc * xc, (((0,), (0,)), ((), ())), preferred_element_type=_f32)


def _gn3_body(xc_ref, b_ref, v_ref, c_ref, g_ref, be_ref, o_ref):
    xc = xc_ref[...]
    oh = (b_ref[...] == lax.broadcasted_iota(jnp.int32, (xc.shape[0], NG), 1)).astype(_f32)
    cnt = c_ref[...].reshape(NG, 1) + 1e-6
    var = v_ref[...] / cnt
    scale = g_ref[...] / jnp.sqrt(var + 1e-5)
    o_ref[...] = xc * jnp.dot(oh, scale, preferred_element_type=_f32) + be_ref[...]


def _gnorm(p, scat, skip, batch):
    s, c = pl.pallas_call(
        _gn1_body,
        grid=(N // NRB,),
        in_specs=[_rows(NRB, H), _rows(NRB, H), _rows(NRB, 1)],
        out_specs=[pl.BlockSpec((NG, H), lambda i: (0, 0)), pl.BlockSpec((1, NG), lambda i: (0, 0))],
        out_shape=[jax.ShapeDtypeStruct((NG, H), _f32), jax.ShapeDtypeStruct((1, NG), _f32)],
    )(scat, skip, batch)
    xc, v = pl.pallas_call(
        _gn2_body,
        grid=(N // NRB,),
        in_specs=[_rows(NRB, H), _rows(NRB, H), _rows(NRB, 1), _full((NG, H)), _full((1, NG)),
                  _full((1, H))],
        out_specs=[_rows(NRB, H), pl.BlockSpec((NG, H), lambda i: (0, 0))],
        out_shape=[jax.ShapeDtypeStruct((N, H), _f32), jax.ShapeDtypeStruct((NG, H), _f32)],
    )(scat, skip, batch, s, c, p["alpha"].reshape(1, H))
    return pl.pallas_call(
        _gn3_body,
        grid=(N // NRB,),
        in_specs=[_rows(NRB, H), _rows(NRB, 1), _full((NG, H)), _full((1, NG)),
                  _full((1, H)), _full((1, H))],
        out_specs=_rows(NRB, H),
        out_shape=jax.ShapeDtypeStruct((N, H), _f32),
    )(xc, batch, v, c, p["gamma"].reshape(1, H), p["beta"].reshape(1, H))


def _tconv(p, x, src, dst, ee, gather_qkv, seg_softmax_msg, scatter_rows):
    q, k, v, skip = _qkvs(p, x)
    eep = _eproj(p, ee)
    qd, ks, vs = gather_qkv(q, k, v, dst, src)
    alpha, gmax = _alpha(qd, ks, eep)
    a = seg_softmax_msg(alpha, gmax, dst)
    msg = _msg(a, vs, eep)
    scat = scatter_rows(msg, dst)
    return scat, skip, msg


def _gather_qkv_jnp(q, k, v, dst, src):
    return q[dst], k[src], v[src]


def _seg_softmax_jnp(alpha, gmax, dst):
    ex = jnp.exp(alpha[:, 0] - gmax[0, 0])
    denom = jax.ops.segment_sum(ex, dst, num_segments=N)
    a = ex / (denom[dst] + 1e-16)
    return a[:, None]


def _scatter_rows_jnp(msg, dst):
    return jax.ops.segment_sum(msg, dst, num_segments=N)


def kernel(x1, edge_index1, edge_attr1, batch1, x2, edge_index2, edge_attr2, batch2, t_value, params):
    pad_e = EP - E
    batch1c = batch1.reshape(N, 1).astype(jnp.int32)
    batch2c = batch2.reshape(N, 1).astype(jnp.int32)
    x2c = x2.reshape(N, 1).astype(jnp.int32)

    ea1 = jnp.pad(edge_attr1[:, 0:1], ((0, pad_e), (0, 0)))
    ea2 = jnp.pad(edge_attr2[:, 0:1], ((0, pad_e), (0, 0)))
    s1 = jnp.pad(edge_index1[0], (0, pad_e)).astype(jnp.int32)
    d1 = jnp.pad(edge_index1[1], (0, pad_e)).astype(jnp.int32)
    s2 = jnp.pad(edge_index2[0], (0, pad_e)).astype(jnp.int32)
    d2 = jnp.pad(edge_index2[1], (0, pad_e)).astype(jnp.int32)

    t_enc, node_enc = _nodeenc(params, x2c, batch1c, t_value)
    nf = jnp.concatenate([t_enc, node_enc], axis=1)

    ee1 = _mlp3(params["edge_full"], ea1, ERB)
    ee1_init = ee1
    ee2 = _mlp3(params["edge_partial"], ea2, ERB)
    ee2_init = ee2

    gq, ss, sr = _gather_qkv_jnp, _seg_softmax_jnp, _scatter_rows_jnp
    for i in range(NB_LAYER):
        sc1, sk1, m1 = _tconv(params["gnn_g"][i], nf, s1, d1, ee1, gq, ss, sr)
        o1 = _gnorm(params["gn_f"][i], sc1, sk1, batch1c)
        sc2, sk2, m2 = _tconv(params["gnn_f"][i], nf, s2, d2, ee2, gq, ss, sr)
        o2 = _gnorm(params["gn_p"][i], sc2, sk2, batch2c)
        ee1 = m1
        ee2 = _mlp3(params["inter"][i], m2, ERB) + ee2_init
        nf = jnp.concatenate([o1, o2], axis=1)

    dec_in = jnp.concatenate([ee1, ee1_init], axis=1)
    out = _mlp3(params["dec"], dec_in, ERB)
    return out[:E]


# speedup vs baseline: 3.6845x; 3.6845x over previous
import functools

import jax
import jax.numpy as jnp
from jax import lax
from jax.experimental import pallas as pl
from jax.experimental.pallas import tpu as pltpu
from jax.experimental.pallas import tpu_sc as plsc

H = 64
NG = 64
MAX_DEG = 64
NB_LAYER = 2
N = 50000
E = 800000
EP = 819200
ERB = 4096
NRB = 2000

_f32 = jnp.float32
_bf16 = jnp.bfloat16


def _bdot(a, b):
    return lax.dot_general(a.astype(_bf16), b.astype(_bf16),
                           (((1,), (0,)), ((), ())), preferred_element_type=_f32)


def _bmul(x, w):
    return x * w


def _full(shape):
    return pl.BlockSpec(shape, lambda i: tuple(0 for _ in shape))


def _rows(rb, c):
    return pl.BlockSpec((rb, c), lambda i: (i, 0))


def _mlp3_body(x_ref, w0_ref, b0_ref, w1_ref, b1_ref, w2_ref, b2_ref, o_ref):
    x = x_ref[...]
    if x.shape[1] > w0_ref.shape[0]:
        x = x[:, :w0_ref.shape[0]]
    if x.shape[1] == 1:
        h = jnp.maximum(_bmul(x, w0_ref[...]) + b0_ref[...], 0.0)
    else:
        h = jnp.maximum(_bdot(x, w0_ref[...]) + b0_ref[...], 0.0)
    h = jnp.maximum(_bdot(h, w1_ref[...]) + b1_ref[...], 0.0)
    o_ref[...] = _bdot(h, w2_ref[...]) + b2_ref[...]


def _mlp3res_body(x_ref, r_ref, w0_ref, b0_ref, w1_ref, b1_ref, w2_ref, b2_ref, o_ref):
    x = x_ref[...]
    if x.shape[1] > w0_ref.shape[0]:
        x = x[:, :w0_ref.shape[0]]
    h = jnp.maximum(_bdot(x, w0_ref[...]) + b0_ref[...], 0.0)
    h = jnp.maximum(_bdot(h, w1_ref[...]) + b1_ref[...], 0.0)
    o_ref[...] = _bdot(h, w2_ref[...]) + b2_ref[...] + r_ref[...]


def _mlp3res(p, x, res, rb):
    r, cin = x.shape
    cout = p["W"][2].shape[1]
    ws = [p["W"][0], p["b"][0].reshape(1, -1), p["W"][1], p["b"][1].reshape(1, -1),
          p["W"][2], p["b"][2].reshape(1, -1)]
    return pl.pallas_call(
        _mlp3res_body,
        grid=(r // rb,),
        in_specs=[_rows(rb, cin), _rows(rb, cout)] + [_full(w.shape) for w in ws],
        out_specs=_rows(rb, cout),
        out_shape=jax.ShapeDtypeStruct((r, cout), _f32),
    )(x, res, *ws)


def _dec_body(m_ref, e_ref, w0a, w0b, b0_ref, w1_ref, b1_ref, w2_ref, b2_ref, o_ref):
    h = jnp.maximum(_bdot(m_ref[:, :H], w0a[...]) + _bdot(e_ref[...], w0b[...]) + b0_ref[...], 0.0)
    h = jnp.maximum(_bdot(h, w1_ref[...]) + b1_ref[...], 0.0)
    o_ref[...] = _bdot(h, w2_ref[...]) + b2_ref[...]


def _dec(p, m128, eei):
    ws = [p["W"][0][:H], p["W"][0][H:], p["b"][0].reshape(1, -1),
          p["W"][1], p["b"][1].reshape(1, -1), p["W"][2], p["b"][2].reshape(1, -1)]
    return pl.pallas_call(
        _dec_body,
        grid=(EP // ERB,),
        in_specs=[_rows(ERB, 2 * H), _rows(ERB, H)] + [_full(w.shape) for w in ws],
        out_specs=_rows(ERB, 1),
        out_shape=jax.ShapeDtypeStruct((EP, 1), _f32),
    )(m128, eei, *ws)


def _mlp3(p, x, rb):
    r, cin = x.shape
    cout = p["W"][2].shape[1]
    ws = [p["W"][0], p["b"][0].reshape(1, -1), p["W"][1], p["b"][1].reshape(1, -1),
          p["W"][2], p["b"][2].reshape(1, -1)]
    return pl.pallas_call(
        _mlp3_body,
        grid=(r // rb,),
        in_specs=[_rows(rb, cin)] + [_full(w.shape) for w in ws],
        out_specs=_rows(rb, cout),
        out_shape=jax.ShapeDtypeStruct((r, cout), _f32),
    )(x, *ws)


def _qkvs_body(x_ref, wq, bq, wkv, bkv, ws, bs, q_ref, kv_ref, s_ref):
    x = x_ref[...]
    q_ref[...] = _bdot(x, wq[...]) + bq[...]
    kv_ref[...] = _bdot(x, wkv[...]) + bkv[...]
    s_ref[...] = _bdot(x, ws[...]) + bs[...]


def _qkvs(p, x):
    wq2 = jnp.pad(p["Wq"], ((0, 0), (0, H)))
    bq2 = jnp.pad(p["bq"], (0, H)).reshape(1, -1)
    wkv = jnp.concatenate([p["Wk"], p["Wv"]], axis=1)
    bkv = jnp.concatenate([p["bk"], p["bv"]]).reshape(1, -1)
    ws = [wq2, bq2, wkv, bkv, p["Wskip"], p["bskip"].reshape(1, -1)]
    o128 = jax.ShapeDtypeStruct((N, 2 * H), _f32)
    o64 = jax.ShapeDtypeStruct((N, H), _f32)
    return pl.pallas_call(
        _qkvs_body,
        grid=(N // NRB,),
        in_specs=[_rows(NRB, 2 * H)] + [_full(w.shape) for w in ws],
        out_specs=[_rows(NRB, 2 * H), _rows(NRB, 2 * H), _rows(NRB, H)],
        out_shape=[o128, o128, o64],
    )(x, *ws)


def _eproj_body(e_ref, w_ref, b_ref, o_ref):
    e = e_ref[...]
    if e.shape[1] > H:
        e = e[:, :H]
    o_ref[...] = _bdot(e, w_ref[...]) + b_ref[...]


def _eproj(p, ee):
    ws = [p["We"], p["be"].reshape(1, -1)]
    return pl.pallas_call(
        _eproj_body,
        grid=(EP // ERB,),
        in_specs=[_rows(ERB, ee.shape[1])] + [_full(w.shape) for w in ws],
        out_specs=_rows(ERB, H),
        out_shape=jax.ShapeDtypeStruct((EP, H), _f32),
    )(ee, *ws)


def _nodeenc_body(x2_ref, b_ref, demb_ref, tval_ref,
                  tw0, tb0, tw1, tb1, tw2, tb2,
                  nw0, nb0, nw1, nb1, nw2, nb2,
                  te_ref, ne_ref):
    deg = jnp.clip(x2_ref[...], 0, MAX_DEG)
    oh_d = (deg == lax.broadcasted_iota(jnp.int32, (deg.shape[0], MAX_DEG + 1), 1)).astype(_f32)
    demb = jnp.dot(oh_d, demb_ref[...], preferred_element_type=_f32, precision=lax.Precision.HIGHEST)
    oh_b = (b_ref[...] == lax.broadcasted_iota(jnp.int32, (deg.shape[0], NG), 1)).astype(_f32)
    tn = jnp.dot(oh_b, tval_ref[...], preferred_element_type=_f32, precision=lax.Precision.HIGHEST)
    h = jnp.maximum(_bmul(tn, tw0[...]) + tb0[...], 0.0)
    h = jnp.maximum(_bdot(h, tw1[...]) + tb1[...], 0.0)
    te_ref[...] = _bdot(h, tw2[...]) + tb2[...]
    g = jnp.maximum(_bdot(demb, nw0[...]) + nb0[...], 0.0)
    g = jnp.maximum(_bdot(g, nw1[...]) + nb1[...], 0.0)
    ne_ref[...] = _bdot(g, nw2[...]) + nb2[...]


def _nodeenc(params, x2, batch1, t_value):
    tp, np_ = params["time"], params["node"]
    ws = [params["deg_emb"], t_value.reshape(NG, 1),
          tp["W"][0], tp["b"][0].reshape(1, -1), tp["W"][1], tp["b"][1].reshape(1, -1),
          tp["W"][2], tp["b"][2].reshape(1, -1),
          np_["W"][0], np_["b"][0].reshape(1, -1), np_["W"][1], np_["b"][1].reshape(1, -1),
          np_["W"][2], np_["b"][2].reshape(1, -1)]
    out = jax.ShapeDtypeStruct((N, H), _f32)
    return pl.pallas_call(
        _nodeenc_body,
        grid=(N // NRB,),
        in_specs=[_rows(NRB, 1), _rows(NRB, 1)] + [_full(w.shape) for w in ws],
        out_specs=[_rows(NRB, H)] * 2,
        out_shape=[out] * 2,
    )(x2, batch1, *ws)


def _alpha_body(qd_ref, kvs_ref, ee_ref, a_ref, gm_ref):
    i = pl.program_id(0)
    ke = kvs_ref[:, :H] + ee_ref[...]
    s = jnp.sum(qd_ref[:, :H] * ke, axis=1, keepdims=True) * 0.125
    rid = i * ERB + lax.broadcasted_iota(jnp.int32, (ERB, 1), 0)
    s = jnp.where(rid < E, s, -1e30)
    a_ref[...] = s
    bm = jnp.max(s, axis=(0, 1), keepdims=True)

    @pl.when(i == 0)
    def _():
        gm_ref[...] = jnp.full((1, 1), -1e30, _f32)

    gm_ref[...] = jnp.maximum(gm_ref[...], bm)


def _alpha(qd, kvs, ee):
    return pl.pallas_call(
        _alpha_body,
        grid=(EP // ERB,),
        in_specs=[_rows(ERB, 2 * H), _rows(ERB, 2 * H), _rows(ERB, H)],
        out_specs=[_rows(ERB, 1), pl.BlockSpec((1, 1), lambda i: (0, 0))],
        out_shape=[jax.ShapeDtypeStruct((EP, 1), _f32), jax.ShapeDtypeStruct((1, 1), _f32)],
    )(qd, kvs, ee)


def _msg_body(a_ref, kvs_ref, ee_ref, o_ref):
    m = a_ref[...] * (kvs_ref[:, H:] + ee_ref[...])
    o_ref[...] = jnp.concatenate([m, m], axis=1)


def _msg(a, kvs, ee):
    return pl.pallas_call(
        _msg_body,
        grid=(EP // ERB,),
        in_specs=[_rows(ERB, 1), _rows(ERB, 2 * H), _rows(ERB, H)],
        out_specs=_rows(ERB, 2 * H),
        out_shape=jax.ShapeDtypeStruct((EP, 2 * H), _f32),
    )(a, kvs, ee)


def _gn1_body(sc_ref, sk_ref, b_ref, s_ref, c_ref):
    i = pl.program_id(0)
    x = sc_ref[:, :H] + sk_ref[...]
    oh = (b_ref[...] == lax.broadcasted_iota(jnp.int32, (x.shape[0], NG), 1)).astype(_f32)

    @pl.when(i == 0)
    def _():
        s_ref[...] = jnp.zeros_like(s_ref)
        c_ref[...] = jnp.zeros_like(c_ref)

    s_ref[...] += lax.dot_general(oh, x, (((0,), (0,)), ((), ())), preferred_element_type=_f32, precision=lax.Precision.HIGHEST)
    c_ref[...] += jnp.sum(oh, axis=0, keepdims=True)


def _gn2_body(sc_ref, sk_ref, b_ref, s_ref, c_ref, al_ref, xc_ref, v_ref):
    i = pl.program_id(0)
    x = sc_ref[:, :H] + sk_ref[...]
    oh = (b_ref[...] == lax.broadcasted_iota(jnp.int32, (x.shape[0], NG), 1)).astype(_f32)
    cnt = c_ref[...].reshape(NG, 1) + 1e-6
    mean = s_ref[...] / cnt
    xc = x - al_ref[...] * jnp.dot(oh, mean, preferred_element_type=_f32, precision=lax.Precision.HIGHEST)
    xc_ref[...] = xc

    @pl.when(i == 0)
    def _():
        v_ref[...] = jnp.zeros_like(v_ref)

    v_ref[...] += lax.dot_general(oh, xc * xc, (((0,), (0,)), ((), ())), preferred_element_type=_f32, precision=lax.Precision.HIGHEST)


def _gn3_body(xc_ref, b_ref, v_ref, c_ref, g_ref, be_ref, o_ref):
    xc = xc_ref[...]
    oh = (b_ref[...] == lax.broadcasted_iota(jnp.int32, (xc.shape[0], NG), 1)).astype(_f32)
    cnt = c_ref[...].reshape(NG, 1) + 1e-6
    var = v_ref[...] / cnt
    scale = g_ref[...] / jnp.sqrt(var + 1e-5)
    o_ref[...] = xc * jnp.dot(oh, scale, preferred_element_type=_f32, precision=lax.Precision.HIGHEST) + be_ref[...]


def _gnorm(p, scat, skip, batch):
    s, c = pl.pallas_call(
        _gn1_body,
        grid=(N // NRB,),
        in_specs=[_rows(NRB, 2 * H), _rows(NRB, H), _rows(NRB, 1)],
        out_specs=[pl.BlockSpec((NG, H), lambda i: (0, 0)), pl.BlockSpec((1, NG), lambda i: (0, 0))],
        out_shape=[jax.ShapeDtypeStruct((NG, H), _f32), jax.ShapeDtypeStruct((1, NG), _f32)],
    )(scat, skip, batch)
    xc, v = pl.pallas_call(
        _gn2_body,
        grid=(N // NRB,),
        in_specs=[_rows(NRB, 2 * H), _rows(NRB, H), _rows(NRB, 1), _full((NG, H)), _full((1, NG)),
                  _full((1, H))],
        out_specs=[_rows(NRB, H), pl.BlockSpec((NG, H), lambda i: (0, 0))],
        out_shape=[jax.ShapeDtypeStruct((N, H), _f32), jax.ShapeDtypeStruct((NG, H), _f32)],
    )(scat, skip, batch, s, c, p["alpha"].reshape(1, H))
    return pl.pallas_call(
        _gn3_body,
        grid=(N // NRB,),
        in_specs=[_rows(NRB, H), _rows(NRB, 1), _full((NG, H)), _full((1, NG)),
                  _full((1, H)), _full((1, H))],
        out_specs=_rows(NRB, H),
        out_shape=jax.ShapeDtypeStruct((N, H), _f32),
    )(xc, batch, v, c, p["gamma"].reshape(1, H), p["beta"].reshape(1, H))


ER = EP // 128
CHW = ER // 32
CHT = ER // 16
NQ = 12544
NQP = NQ + 88


def _sc_mesh():
    return plsc.VectorSubcoreMesh(core_axis_name="c", subcore_axis_name="s")


def _sc_gather2(q2, kv, dst2d, src2d):
    out = jax.ShapeDtypeStruct((EP, 2 * H), _f32)

    @functools.partial(
        pl.kernel, mesh=_sc_mesh(), out_type=[out, out],
        scratch_types=[pltpu.VMEM((CHW, 128), jnp.int32),
                       pltpu.VMEM((CHW, 128), jnp.int32),
                       pltpu.VMEM((128, 2 * H), _f32),
                       pltpu.VMEM((128, 2 * H), _f32),
                       pltpu.SemaphoreType.DMA,
                       pltpu.SemaphoreType.DMA],
    )
    def kfn(q_hbm, kv_hbm, di_hbm, si_hbm, qd_hbm, kvs_hbm,
            di_v, si_v, bq, bkv, sq, skv):
        wid = lax.axis_index("s") * 2 + lax.axis_index("c")
        rbase = wid * CHW
        pltpu.sync_copy(di_hbm.at[pl.ds(rbase, CHW)], di_v)
        pltpu.sync_copy(si_hbm.at[pl.ds(rbase, CHW)], si_v)

        def body(j, carry):
            cq = pltpu.async_copy(q_hbm.at[di_v.at[j]], bq, sq)
            ckv = pltpu.async_copy(kv_hbm.at[si_v.at[j]], bkv, skv)
            cq.wait()
            ckv.wait()
            ebase = (rbase + j) * 128
            pltpu.sync_copy(bq, qd_hbm.at[pl.ds(ebase, 128)])
            pltpu.sync_copy(bkv, kvs_hbm.at[pl.ds(ebase, 128)])
            return carry

        lax.fori_loop(0, CHW, body, 0)

    return kfn(q2, kv, dst2d, src2d)


def _sc_denom(alpha2d, dst2d, gmax16, zeros_n):
    @functools.partial(
        pl.kernel, mesh=_sc_mesh(), out_type=jax.ShapeDtypeStruct((N,), _f32),
        scratch_types=[pltpu.VMEM((CHT, 128), _f32),
                       pltpu.VMEM((CHT, 128), jnp.int32),
                       pltpu.VMEM((16,), _f32),
                       pltpu.VMEM_SHARED((N,), _f32)],
    )
    def kfn(al_hbm, di_hbm, gm_hbm, z_hbm, den_hbm, al_v, di_v, gm_v, acc_sh):
        c = lax.axis_index("c")
        s = lax.axis_index("s")

        @pl.when(c == 0)
        def _():
            @pl.when(s == 0)
            def _z():
                pltpu.sync_copy(z_hbm, acc_sh)

            rbase = s * CHT
            pltpu.sync_copy(al_hbm.at[pl.ds(rbase, CHT)], al_v)
            pltpu.sync_copy(di_hbm.at[pl.ds(rbase, CHT)], di_v)
            pltpu.sync_copy(gm_hbm, gm_v)
            g = gm_v[...]

            def eb(t, carry):
                j = t // 8
                sl = pl.ds((t % 8) * 16, 16)
                al_v[j, sl] = jnp.exp(al_v[j, sl] - g)
                return carry

            lax.fori_loop(0, CHT * 8, eb, 0)
            plsc.subcore_barrier()

            def sb(j, carry):
                pltpu.sync_copy(al_v.at[j], acc_sh.at[di_v.at[j]], add=True)
                return carry

            lax.fori_loop(0, CHT, sb, 0)
            plsc.subcore_barrier()

            @pl.when(s == 0)
            def _w():
                pltpu.sync_copy(acc_sh, den_hbm)

    return kfn(alpha2d, dst2d, gmax16, zeros_n)


def _sc_adiv(alpha2d, dst2d, gmax16, denom):
    @functools.partial(
        pl.kernel, mesh=_sc_mesh(), out_type=jax.ShapeDtypeStruct((ER, 128), _f32),
        scratch_types=[pltpu.VMEM_SHARED((N,), _f32),
                       pltpu.VMEM((CHW, 128), _f32),
                       pltpu.VMEM((CHW, 128), jnp.int32),
                       pltpu.VMEM((128,), _f32),
                       pltpu.VMEM((16,), _f32)],
    )
    def kfn(al_hbm, di_hbm, gm_hbm, den_hbm, a_hbm, den_sh, al_v, di_v, den_g, gm_v):
        c = lax.axis_index("c")
        s = lax.axis_index("s")
        wid = s * 2 + c
        rbase = wid * CHW

        @pl.when(s == 0)
        def _z():
            pltpu.sync_copy(den_hbm, den_sh)

        pltpu.sync_copy(al_hbm.at[pl.ds(rbase, CHW)], al_v)
        pltpu.sync_copy(di_hbm.at[pl.ds(rbase, CHW)], di_v)
        pltpu.sync_copy(gm_hbm, gm_v)
        g = gm_v[...]
        plsc.subcore_barrier()

        def jb(j, carry):
            pltpu.sync_copy(den_sh.at[di_v.at[j]], den_g)
            for l in range(8):
                sl = pl.ds(l * 16, 16)
                al_v[j, sl] = jnp.exp(al_v[j, sl] - g) / (den_g[sl] + 1e-16)
            return carry

        lax.fori_loop(0, CHW, jb, 0)
        pltpu.sync_copy(al_v, a_hbm.at[pl.ds(rbase, CHW)])

    return kfn(alpha2d, dst2d, gmax16, denom)


def _sc_scatter_rows(msg128, dst2d):
    @functools.partial(
        pl.kernel, mesh=_sc_mesh(), out_type=jax.ShapeDtypeStruct((N, 2 * H), _f32),
        scratch_types=[pltpu.VMEM((40, 128), jnp.int32),
                       pltpu.VMEM((128, 2 * H), _f32),
                       pltpu.VMEM_SHARED((NQP, 2 * H), _f32)],
    )
    def kfn(m_hbm, di_hbm, out_hbm, di_v, mb, acc_sh):
        c = lax.axis_index("c")
        s = lax.axis_index("s")
        rbase = s * CHT
        dump0 = lax.iota(jnp.int32, 16) * 4 + NQ

        for r in range(2):
            base_node = (c * 2 + r) * NQ

            def zvec(t, carry):
                mb[t // 8, pl.ds((t % 8) * 16, 16)] = jnp.zeros((16,), _f32)
                return carry

            lax.fori_loop(0, 128 * 8, zvec, 0)

            def zc(i, carry):
                pltpu.sync_copy(mb, acc_sh.at[pl.ds(i * 128, 128)])
                return carry

            lax.fori_loop(s * 7, jnp.minimum(s * 7 + 7, 98), zc, 0)

            @pl.when(s == 0)
            def _ztail():
                pltpu.sync_copy(mb.at[pl.ds(0, NQP - 98 * 128)],
                                acc_sh.at[pl.ds(98 * 128, NQP - 98 * 128)])

            plsc.subcore_barrier()

            def bb(b, carry):
                pltpu.sync_copy(di_hbm.at[pl.ds(rbase + b * 40, 40)], di_v)

                def tb(t, c2):
                    j = t // 8
                    sl = pl.ds((t % 8) * 16, 16)
                    ix = di_v[j, sl] - base_node
                    inb = (ix >= 0) & (ix < NQ)
                    di_v[j, sl] = jnp.where(inb, ix, dump0 + (t % 4))
                    return c2

                lax.fori_loop(0, 40 * 8, tb, 0)

                def sb(j, c2):
                    pltpu.sync_copy(m_hbm.at[pl.ds((rbase + b * 40 + j) * 128, 128)], mb)
                    pltpu.sync_copy(mb, acc_sh.at[di_v.at[j]], add=True)
                    return c2

                lax.fori_loop(0, 40, sb, 0)
                return carry

            lax.fori_loop(0, CHT // 40, bb, 0)
            plsc.subcore_barrier()

            @pl.when(s == 0)
            def _w():
                @pl.when(c * 2 + r < 3)
                def _full_():
                    pltpu.sync_copy(acc_sh.at[pl.ds(0, NQ)],
                                    out_hbm.at[pl.ds(base_node, NQ)])

                @pl.when(c * 2 + r == 3)
                def _tail_():
                    pltpu.sync_copy(acc_sh.at[pl.ds(0, N - 3 * NQ)],
                                    out_hbm.at[pl.ds(3 * NQ, N - 3 * NQ)])

            plsc.subcore_barrier()

    return kfn(msg128, dst2d)


def _tconv(p, x, dst2d, src2d, ee, zeros_n):
    q2, kv, skip = _qkvs(p, x)
    eep = _eproj(p, ee)
    qd, kvs = _sc_gather2(q2, kv, dst2d, src2d)
    alpha, gmax = _alpha(qd, kvs, eep)
    alpha2d = alpha.reshape(ER, 128)
    gmax16 = jnp.broadcast_to(gmax.reshape(1), (16,))
    denom = _sc_denom(alpha2d, dst2d, gmax16, zeros_n)
    a2d = _sc_adiv(alpha2d, dst2d, gmax16, denom)
    msg = _msg(a2d.reshape(EP, 1), kvs, eep)
    scat = _sc_scatter_rows(msg, dst2d)
    return scat, skip, msg


def kernel(x1, edge_index1, edge_attr1, batch1, x2, edge_index2, edge_attr2, batch2, t_value, params):
    pad_e = EP - E
    batch1c = batch1.reshape(N, 1).astype(jnp.int32)
    batch2c = batch2.reshape(N, 1).astype(jnp.int32)
    x2c = x2.reshape(N, 1).astype(jnp.int32)

    ea1 = jnp.pad(edge_attr1[:, 0:1], ((0, pad_e), (0, 0)))
    ea2 = jnp.pad(edge_attr2[:, 0:1], ((0, pad_e), (0, 0)))
    s1 = jnp.pad(edge_index1[0], (0, pad_e)).astype(jnp.int32).reshape(ER, 128)
    d1 = jnp.pad(edge_index1[1], (0, pad_e)).astype(jnp.int32).reshape(ER, 128)
    s2 = jnp.pad(edge_index2[0], (0, pad_e)).astype(jnp.int32).reshape(ER, 128)
    d2 = jnp.pad(edge_index2[1], (0, pad_e)).astype(jnp.int32).reshape(ER, 128)
    zeros_n = jnp.zeros((N,), _f32)

    t_enc, node_enc = _nodeenc(params, x2c, batch1c, t_value)
    nf = jnp.concatenate([t_enc, node_enc], axis=1)

    ee1 = _mlp3(params["edge_full"], ea1, ERB)
    ee1_init = ee1
    ee2 = _mlp3(params["edge_partial"], ea2, ERB)
    ee2_init = ee2

    for i in range(NB_LAYER):
        sc1, sk1, m1 = _tconv(params["gnn_g"][i], nf, d1, s1, ee1, zeros_n)
        o1 = _gnorm(params["gn_f"][i], sc1, sk1, batch1c)
        d2b = lax.optimization_barrier((d2, sc1))[0]
        sc2, sk2, m2 = _tconv(params["gnn_f"][i], nf, d2b, s2, ee2, zeros_n)
        o2 = _gnorm(params["gn_p"][i], sc2, sk2, batch2c)
        ee1 = m1
        ee2 = _mlp3res(params["inter"][i], m2, ee2_init, ERB)
        nf = jnp.concatenate([o1, o2], axis=1)

    out = _dec(params["dec"], ee1, ee1_init)
    return out[:E]


# speedup vs baseline: 3.7935x; 1.0296x over previous
import functools

import jax
import jax.numpy as jnp
from jax import lax
from jax.experimental import pallas as pl
from jax.experimental.pallas import tpu as pltpu
from jax.experimental.pallas import tpu_sc as plsc

H = 64
NG = 64
MAX_DEG = 64
NB_LAYER = 2
N = 50000
E = 800000
EP = 819200
ERB = 4096
NRB = 2000

_f32 = jnp.float32
_bf16 = jnp.bfloat16


def _bdot(a, b):
    return lax.dot_general(a.astype(_bf16), b.astype(_bf16),
                           (((1,), (0,)), ((), ())), preferred_element_type=_f32)


def _bmul(x, w):
    return x * w


def _full(shape):
    return pl.BlockSpec(shape, lambda i: tuple(0 for _ in shape))


def _rows(rb, c):
    return pl.BlockSpec((rb, c), lambda i: (i, 0))


def _mlp3_body(x_ref, w0_ref, b0_ref, w1_ref, b1_ref, w2_ref, b2_ref, o_ref):
    x = x_ref[...]
    if x.shape[1] > w0_ref.shape[0]:
        x = x[:, :w0_ref.shape[0]]
    if x.shape[1] == 1:
        h = jnp.maximum(_bmul(x, w0_ref[...]) + b0_ref[...], 0.0)
    else:
        h = jnp.maximum(_bdot(x, w0_ref[...]) + b0_ref[...], 0.0)
    h = jnp.maximum(_bdot(h, w1_ref[...]) + b1_ref[...], 0.0)
    o_ref[...] = _bdot(h, w2_ref[...]) + b2_ref[...]


def _mlp3res_body(x_ref, r_ref, w0_ref, b0_ref, w1_ref, b1_ref, w2_ref, b2_ref, o_ref):
    x = x_ref[...]
    if x.shape[1] > w0_ref.shape[0]:
        x = x[:, :w0_ref.shape[0]]
    h = jnp.maximum(_bdot(x, w0_ref[...]) + b0_ref[...], 0.0)
    h = jnp.maximum(_bdot(h, w1_ref[...]) + b1_ref[...], 0.0)
    o_ref[...] = _bdot(h, w2_ref[...]) + b2_ref[...] + r_ref[...]


def _mlp3res(p, x, res, rb):
    r, cin = x.shape
    cout = p["W"][2].shape[1]
    ws = [p["W"][0], p["b"][0].reshape(1, -1), p["W"][1], p["b"][1].reshape(1, -1),
          p["W"][2], p["b"][2].reshape(1, -1)]
    return pl.pallas_call(
        _mlp3res_body,
        grid=(r // rb,),
        in_specs=[_rows(rb, cin), _rows(rb, cout)] + [_full(w.shape) for w in ws],
        out_specs=_rows(rb, cout),
        out_shape=jax.ShapeDtypeStruct((r, cout), _f32),
    )(x, res, *ws)


def _dec_body(m_ref, e_ref, w0a, w0b, b0_ref, w1_ref, b1_ref, w2_ref, b2_ref, o_ref):
    h = jnp.maximum(_bdot(m_ref[:, :H], w0a[...]) + _bdot(e_ref[...], w0b[...]) + b0_ref[...], 0.0)
    h = jnp.maximum(_bdot(h, w1_ref[...]) + b1_ref[...], 0.0)
    o_ref[...] = _bdot(h, w2_ref[...]) + b2_ref[...]


def _dec(p, m128, eei):
    ws = [p["W"][0][:H], p["W"][0][H:], p["b"][0].reshape(1, -1),
          p["W"][1], p["b"][1].reshape(1, -1), p["W"][2], p["b"][2].reshape(1, -1)]
    return pl.pallas_call(
        _dec_body,
        grid=(EP // ERB,),
        in_specs=[_rows(ERB, 2 * H), _rows(ERB, H)] + [_full(w.shape) for w in ws],
        out_specs=_rows(ERB, 1),
        out_shape=jax.ShapeDtypeStruct((EP, 1), _f32),
    )(m128, eei, *ws)


def _mlp3(p, x, rb):
    r, cin = x.shape
    cout = p["W"][2].shape[1]
    ws = [p["W"][0], p["b"][0].reshape(1, -1), p["W"][1], p["b"][1].reshape(1, -1),
          p["W"][2], p["b"][2].reshape(1, -1)]
    return pl.pallas_call(
        _mlp3_body,
        grid=(r // rb,),
        in_specs=[_rows(rb, cin)] + [_full(w.shape) for w in ws],
        out_specs=_rows(rb, cout),
        out_shape=jax.ShapeDtypeStruct((r, cout), _f32),
    )(x, *ws)


def _qkvs_body(x_ref, wq, bq, wkv, bkv, ws, bs, q_ref, kv_ref, s_ref):
    x = x_ref[...]
    q_ref[...] = _bdot(x, wq[...]) + bq[...]
    kv_ref[...] = _bdot(x, wkv[...]) + bkv[...]
    s_ref[...] = _bdot(x, ws[...]) + bs[...]


def _qkvs(p, x):
    wq2 = jnp.pad(p["Wq"], ((0, 0), (0, H)))
    bq2 = jnp.pad(p["bq"], (0, H)).reshape(1, -1)
    wkv = jnp.concatenate([p["Wk"], p["Wv"]], axis=1)
    bkv = jnp.concatenate([p["bk"], p["bv"]]).reshape(1, -1)
    ws = [wq2, bq2, wkv, bkv, p["Wskip"], p["bskip"].reshape(1, -1)]
    o128 = jax.ShapeDtypeStruct((N, 2 * H), _f32)
    o64 = jax.ShapeDtypeStruct((N, H), _f32)
    return pl.pallas_call(
        _qkvs_body,
        grid=(N // NRB,),
        in_specs=[_rows(NRB, 2 * H)] + [_full(w.shape) for w in ws],
        out_specs=[_rows(NRB, 2 * H), _rows(NRB, 2 * H), _rows(NRB, H)],
        out_shape=[o128, o128, o64],
    )(x, *ws)


def _eproj_body(e_ref, w_ref, b_ref, o_ref):
    e = e_ref[...]
    if e.shape[1] > H:
        e = e[:, :H]
    o_ref[...] = _bdot(e, w_ref[...]) + b_ref[...]


def _eproj(p, ee):
    ws = [p["We"], p["be"].reshape(1, -1)]
    return pl.pallas_call(
        _eproj_body,
        grid=(EP // ERB,),
        in_specs=[_rows(ERB, ee.shape[1])] + [_full(w.shape) for w in ws],
        out_specs=_rows(ERB, H),
        out_shape=jax.ShapeDtypeStruct((EP, H), _f32),
    )(ee, *ws)


def _nodeenc_body(x2_ref, b_ref, demb_ref, tval_ref,
                  tw0, tb0, tw1, tb1, tw2, tb2,
                  nw0, nb0, nw1, nb1, nw2, nb2,
                  te_ref, ne_ref):
    deg = jnp.clip(x2_ref[...], 0, MAX_DEG)
    oh_d = (deg == lax.broadcasted_iota(jnp.int32, (deg.shape[0], MAX_DEG + 1), 1)).astype(_f32)
    demb = jnp.dot(oh_d, demb_ref[...], preferred_element_type=_f32, precision=lax.Precision.HIGHEST)
    oh_b = (b_ref[...] == lax.broadcasted_iota(jnp.int32, (deg.shape[0], NG), 1)).astype(_f32)
    tn = jnp.dot(oh_b, tval_ref[...], preferred_element_type=_f32, precision=lax.Precision.HIGHEST)
    h = jnp.maximum(_bmul(tn, tw0[...]) + tb0[...], 0.0)
    h = jnp.maximum(_bdot(h, tw1[...]) + tb1[...], 0.0)
    te_ref[...] = _bdot(h, tw2[...]) + tb2[...]
    g = jnp.maximum(_bdot(demb, nw0[...]) + nb0[...], 0.0)
    g = jnp.maximum(_bdot(g, nw1[...]) + nb1[...], 0.0)
    ne_ref[...] = _bdot(g, nw2[...]) + nb2[...]


def _nodeenc(params, x2, batch1, t_value):
    tp, np_ = params["time"], params["node"]
    ws = [params["deg_emb"], t_value.reshape(NG, 1),
          tp["W"][0], tp["b"][0].reshape(1, -1), tp["W"][1], tp["b"][1].reshape(1, -1),
          tp["W"][2], tp["b"][2].reshape(1, -1),
          np_["W"][0], np_["b"][0].reshape(1, -1), np_["W"][1], np_["b"][1].reshape(1, -1),
          np_["W"][2], np_["b"][2].reshape(1, -1)]
    out = jax.ShapeDtypeStruct((N, H), _f32)
    return pl.pallas_call(
        _nodeenc_body,
        grid=(N // NRB,),
        in_specs=[_rows(NRB, 1), _rows(NRB, 1)] + [_full(w.shape) for w in ws],
        out_specs=[_rows(NRB, H)] * 2,
        out_shape=[out] * 2,
    )(x2, batch1, *ws)


def _alpha_body(qd_ref, kvs_ref, ee_ref, a_ref, gm_ref):
    i = pl.program_id(0)
    ke = kvs_ref[:, :H] + ee_ref[...]
    s = jnp.sum(qd_ref[:, :H] * ke, axis=1, keepdims=True) * 0.125
    rid = i * ERB + lax.broadcasted_iota(jnp.int32, (ERB, 1), 0)
    s = jnp.where(rid < E, s, -1e30)
    a_ref[...] = s
    bm = jnp.max(s, axis=(0, 1), keepdims=True)

    @pl.when(i == 0)
    def _():
        gm_ref[...] = jnp.full((1, 1), -1e30, _f32)

    gm_ref[...] = jnp.maximum(gm_ref[...], bm)


def _alpha(qd, kvs, ee):
    return pl.pallas_call(
        _alpha_body,
        grid=(EP // ERB,),
        in_specs=[_rows(ERB, 2 * H), _rows(ERB, 2 * H), _rows(ERB, H)],
        out_specs=[_rows(ERB, 1), pl.BlockSpec((1, 1), lambda i: (0, 0))],
        out_shape=[jax.ShapeDtypeStruct((EP, 1), _f32), jax.ShapeDtypeStruct((1, 1), _f32)],
    )(qd, kvs, ee)


def _msg_body(a_ref, kvs_ref, ee_ref, o_ref):
    m = a_ref[...] * (kvs_ref[:, H:] + ee_ref[...])
    o_ref[...] = jnp.concatenate([m, m], axis=1)


def _msg(a, kvs, ee):
    return pl.pallas_call(
        _msg_body,
        grid=(EP // ERB,),
        in_specs=[_rows(ERB, 1), _rows(ERB, 2 * H), _rows(ERB, H)],
        out_specs=_rows(ERB, 2 * H),
        out_shape=jax.ShapeDtypeStruct((EP, 2 * H), _f32),
    )(a, kvs, ee)


def _gn1_body(sc_ref, sk_ref, b_ref, s_ref, c_ref):
    i = pl.program_id(0)
    x = sc_ref[:, :H] + sk_ref[...]
    oh = (b_ref[...] == lax.broadcasted_iota(jnp.int32, (x.shape[0], NG), 1)).astype(_f32)

    @pl.when(i == 0)
    def _():
        s_ref[...] = jnp.zeros_like(s_ref)
        c_ref[...] = jnp.zeros_like(c_ref)

    s_ref[...] += lax.dot_general(oh, x, (((0,), (0,)), ((), ())), preferred_element_type=_f32, precision=lax.Precision.HIGHEST)
    c_ref[...] += jnp.sum(oh, axis=0, keepdims=True)


def _gn2_body(sc_ref, sk_ref, b_ref, s_ref, c_ref, al_ref, xc_ref, v_ref):
    i = pl.program_id(0)
    x = sc_ref[:, :H] + sk_ref[...]
    oh = (b_ref[...] == lax.broadcasted_iota(jnp.int32, (x.shape[0], NG), 1)).astype(_f32)
    cnt = c_ref[...].reshape(NG, 1) + 1e-6
    mean = s_ref[...] / cnt
    xc = x - al_ref[...] * jnp.dot(oh, mean, preferred_element_type=_f32, precision=lax.Precision.HIGHEST)
    xc_ref[...] = xc

    @pl.when(i == 0)
    def _():
        v_ref[...] = jnp.zeros_like(v_ref)

    v_ref[...] += lax.dot_general(oh, xc * xc, (((0,), (0,)), ((), ())), preferred_element_type=_f32, precision=lax.Precision.HIGHEST)


def _gn3_body(xc_ref, b_ref, v_ref, c_ref, g_ref, be_ref, o_ref):
    xc = xc_ref[...]
    oh = (b_ref[...] == lax.broadcasted_iota(jnp.int32, (xc.shape[0], NG), 1)).astype(_f32)
    cnt = c_ref[...].reshape(NG, 1) + 1e-6
    var = v_ref[...] / cnt
    scale = g_ref[...] / jnp.sqrt(var + 1e-5)
    o_ref[...] = xc * jnp.dot(oh, scale, preferred_element_type=_f32, precision=lax.Precision.HIGHEST) + be_ref[...]


def _gnorm(p, scat, skip, batch):
    s, c = pl.pallas_call(
        _gn1_body,
        grid=(N // NRB,),
        in_specs=[_rows(NRB, 2 * H), _rows(NRB, H), _rows(NRB, 1)],
        out_specs=[pl.BlockSpec((NG, H), lambda i: (0, 0)), pl.BlockSpec((1, NG), lambda i: (0, 0))],
        out_shape=[jax.ShapeDtypeStruct((NG, H), _f32), jax.ShapeDtypeStruct((1, NG), _f32)],
    )(scat, skip, batch)
    xc, v = pl.pallas_call(
        _gn2_body,
        grid=(N // NRB,),
        in_specs=[_rows(NRB, 2 * H), _rows(NRB, H), _rows(NRB, 1), _full((NG, H)), _full((1, NG)),
                  _full((1, H))],
        out_specs=[_rows(NRB, H), pl.BlockSpec((NG, H), lambda i: (0, 0))],
        out_shape=[jax.ShapeDtypeStruct((N, H), _f32), jax.ShapeDtypeStruct((NG, H), _f32)],
    )(scat, skip, batch, s, c, p["alpha"].reshape(1, H))
    return pl.pallas_call(
        _gn3_body,
        grid=(N // NRB,),
        in_specs=[_rows(NRB, H), _rows(NRB, 1), _full((NG, H)), _full((1, NG)),
                  _full((1, H)), _full((1, H))],
        out_specs=_rows(NRB, H),
        out_shape=jax.ShapeDtypeStruct((N, H), _f32),
    )(xc, batch, v, c, p["gamma"].reshape(1, H), p["beta"].reshape(1, H))


ER = EP // 128
CHW = ER // 32
CHT = ER // 16
NQ = 12544
NQP = NQ + 88


def _sc_mesh():
    return plsc.VectorSubcoreMesh(core_axis_name="c", subcore_axis_name="s")


def _sc_gather2(q2, kv, dst2d, src2d):
    out = jax.ShapeDtypeStruct((EP, 2 * H), _f32)

    @functools.partial(
        pl.kernel, mesh=_sc_mesh(), out_type=[out, out],
        scratch_types=[pltpu.VMEM((CHW, 128), jnp.int32),
                       pltpu.VMEM((CHW, 128), jnp.int32),
                       pltpu.VMEM((128, 2 * H), _f32),
                       pltpu.VMEM((128, 2 * H), _f32),
                       pltpu.VMEM((128, 2 * H), _f32),
                       pltpu.VMEM((128, 2 * H), _f32),
                       pltpu.SemaphoreType.DMA,
                       pltpu.SemaphoreType.DMA,
                       pltpu.SemaphoreType.DMA,
                       pltpu.SemaphoreType.DMA],
    )
    def kfn(q_hbm, kv_hbm, di_hbm, si_hbm, qd_hbm, kvs_hbm,
            di_v, si_v, bq0, bq1, bkv0, bkv1, sq0, sq1, skv0, skv1):
        wid = lax.axis_index("s") * 2 + lax.axis_index("c")
        rbase = wid * CHW
        pltpu.sync_copy(di_hbm.at[pl.ds(rbase, CHW)], di_v)
        pltpu.sync_copy(si_hbm.at[pl.ds(rbase, CHW)], si_v)
        bq = (bq0, bq1)
        bkv = (bkv0, bkv1)
        sq = (sq0, sq1)
        skv = (skv0, skv1)

        pltpu.async_copy(q_hbm.at[di_v.at[0]], bq[0], sq[0])
        pltpu.async_copy(kv_hbm.at[si_v.at[0]], bkv[0], skv[0])

        def body(p, carry):
            for b in range(2):
                j = 2 * p + b
                pltpu.make_async_copy(q_hbm.at[di_v.at[j]], bq[b], sq[b]).wait()
                pltpu.make_async_copy(kv_hbm.at[si_v.at[j]], bkv[b], skv[b]).wait()
                if b == 0:
                    pltpu.async_copy(q_hbm.at[di_v.at[j + 1]], bq[1], sq[1])
                    pltpu.async_copy(kv_hbm.at[si_v.at[j + 1]], bkv[1], skv[1])
                else:
                    @pl.when(p < CHW // 2 - 1)
                    def _pref():
                        pltpu.async_copy(q_hbm.at[di_v.at[j + 1]], bq[0], sq[0])
                        pltpu.async_copy(kv_hbm.at[si_v.at[j + 1]], bkv[0], skv[0])

                ebase = (rbase + j) * 128
                pltpu.sync_copy(bq[b], qd_hbm.at[pl.ds(ebase, 128)])
                pltpu.sync_copy(bkv[b], kvs_hbm.at[pl.ds(ebase, 128)])
            return carry

        lax.fori_loop(0, CHW // 2, body, 0)

    return kfn(q2, kv, dst2d, src2d)


def _sc_denom(alpha2d, dst2d, gmax16, zeros_n):
    @functools.partial(
        pl.kernel, mesh=_sc_mesh(), out_type=jax.ShapeDtypeStruct((N,), _f32),
        scratch_types=[pltpu.VMEM((CHT, 128), _f32),
                       pltpu.VMEM((CHT, 128), jnp.int32),
                       pltpu.VMEM((16,), _f32),
                       pltpu.VMEM_SHARED((N,), _f32)],
    )
    def kfn(al_hbm, di_hbm, gm_hbm, z_hbm, den_hbm, al_v, di_v, gm_v, acc_sh):
        c = lax.axis_index("c")
        s = lax.axis_index("s")

        @pl.when(c == 0)
        def _():
            @pl.when(s == 0)
            def _z():
                pltpu.sync_copy(z_hbm, acc_sh)

            rbase = s * CHT
            pltpu.sync_copy(al_hbm.at[pl.ds(rbase, CHT)], al_v)
            pltpu.sync_copy(di_hbm.at[pl.ds(rbase, CHT)], di_v)
            pltpu.sync_copy(gm_hbm, gm_v)
            g = gm_v[...]

            def eb(t, carry):
                j = t // 8
                sl = pl.ds((t % 8) * 16, 16)
                al_v[j, sl] = jnp.exp(al_v[j, sl] - g)
                return carry

            lax.fori_loop(0, CHT * 8, eb, 0)
            plsc.subcore_barrier()

            def sb(j, carry):
                pltpu.sync_copy(al_v.at[j], acc_sh.at[di_v.at[j]], add=True)
                return carry

            lax.fori_loop(0, CHT, sb, 0)
            plsc.subcore_barrier()

            @pl.when(s == 0)
            def _w():
                pltpu.sync_copy(acc_sh, den_hbm)

    return kfn(alpha2d, dst2d, gmax16, zeros_n)


def _sc_adiv(alpha2d, dst2d, gmax16, denom):
    @functools.partial(
        pl.kernel, mesh=_sc_mesh(), out_type=jax.ShapeDtypeStruct((ER, 128), _f32),
        scratch_types=[pltpu.VMEM_SHARED((N,), _f32),
                       pltpu.VMEM((CHW, 128), _f32),
                       pltpu.VMEM((CHW, 128), jnp.int32),
                       pltpu.VMEM((128,), _f32),
                       pltpu.VMEM((16,), _f32)],
    )
    def kfn(al_hbm, di_hbm, gm_hbm, den_hbm, a_hbm, den_sh, al_v, di_v, den_g, gm_v):
        c = lax.axis_index("c")
        s = lax.axis_index("s")
        wid = s * 2 + c
        rbase = wid * CHW

        @pl.when(s == 0)
        def _z():
            pltpu.sync_copy(den_hbm, den_sh)

        pltpu.sync_copy(al_hbm.at[pl.ds(rbase, CHW)], al_v)
        pltpu.sync_copy(di_hbm.at[pl.ds(rbase, CHW)], di_v)
        pltpu.sync_copy(gm_hbm, gm_v)
        g = gm_v[...]
        plsc.subcore_barrier()

        def jb(j, carry):
            pltpu.sync_copy(den_sh.at[di_v.at[j]], den_g)
            for l in range(8):
                sl = pl.ds(l * 16, 16)
                al_v[j, sl] = jnp.exp(al_v[j, sl] - g) / (den_g[sl] + 1e-16)
            return carry

        lax.fori_loop(0, CHW, jb, 0)
        pltpu.sync_copy(al_v, a_hbm.at[pl.ds(rbase, CHW)])

    return kfn(alpha2d, dst2d, gmax16, denom)


def _sc_scatter_rows(msg128, dst2d):
    @functools.partial(
        pl.kernel, mesh=_sc_mesh(), out_type=jax.ShapeDtypeStruct((N, 2 * H), _f32),
        scratch_types=[pltpu.VMEM((40, 128), jnp.int32),
                       pltpu.VMEM((128, 2 * H), _f32),
                       pltpu.VMEM_SHARED((NQP, 2 * H), _f32)],
    )
    def kfn(m_hbm, di_hbm, out_hbm, di_v, mb, acc_sh):
        c = lax.axis_index("c")
        s = lax.axis_index("s")
        rbase = s * CHT
        dump0 = lax.iota(jnp.int32, 16) * 4 + NQ

        for r in range(2):
            base_node = (c * 2 + r) * NQ

            def zvec(t, carry):
                mb[t // 8, pl.ds((t % 8) * 16, 16)] = jnp.zeros((16,), _f32)
                return carry

            lax.fori_loop(0, 128 * 8, zvec, 0)

            def zc(i, carry):
                pltpu.sync_copy(mb, acc_sh.at[pl.ds(i * 128, 128)])
                return carry

            lax.fori_loop(s * 7, jnp.minimum(s * 7 + 7, 98), zc, 0)

            @pl.when(s == 0)
            def _ztail():
                pltpu.sync_copy(mb.at[pl.ds(0, NQP - 98 * 128)],
                                acc_sh.at[pl.ds(98 * 128, NQP - 98 * 128)])

            plsc.subcore_barrier()

            def bb(b, carry):
                pltpu.sync_copy(di_hbm.at[pl.ds(rbase + b * 40, 40)], di_v)

                def tb(t, c2):
                    j = t // 8
                    sl = pl.ds((t % 8) * 16, 16)
                    ix = di_v[j, sl] - base_node
                    inb = (ix >= 0) & (ix < NQ)
                    di_v[j, sl] = jnp.where(inb, ix, dump0 + (t % 4))
                    return c2

                lax.fori_loop(0, 40 * 8, tb, 0)

                def sb(j, c2):
                    pltpu.sync_copy(m_hbm.at[pl.ds((rbase + b * 40 + j) * 128, 128)], mb)
                    pltpu.sync_copy(mb, acc_sh.at[di_v.at[j]], add=True)
                    return c2

                lax.fori_loop(0, 40, sb, 0)
                return carry

            lax.fori_loop(0, CHT // 40, bb, 0)
            plsc.subcore_barrier()

            @pl.when(s == 0)
            def _w():
                @pl.when(c * 2 + r < 3)
                def _full_():
                    pltpu.sync_copy(acc_sh.at[pl.ds(0, NQ)],
                                    out_hbm.at[pl.ds(base_node, NQ)])

                @pl.when(c * 2 + r == 3)
                def _tail_():
                    pltpu.sync_copy(acc_sh.at[pl.ds(0, N - 3 * NQ)],
                                    out_hbm.at[pl.ds(3 * NQ, N - 3 * NQ)])

            plsc.subcore_barrier()

    return kfn(msg128, dst2d)


def _tconv(p, x, dst2d, src2d, ee, zeros_n):
    q2, kv, skip = _qkvs(p, x)
    eep = _eproj(p, ee)
    qd, kvs = _sc_gather2(q2, kv, dst2d, src2d)
    alpha, gmax = _alpha(qd, kvs, eep)
    alpha2d = alpha.reshape(ER, 128)
    gmax16 = jnp.broadcast_to(gmax.reshape(1), (16,))
    denom = _sc_denom(alpha2d, dst2d, gmax16, zeros_n)
    a2d = _sc_adiv(alpha2d, dst2d, gmax16, denom)
    msg = _msg(a2d.reshape(EP, 1), kvs, eep)
    scat = _sc_scatter_rows(msg, dst2d)
    return scat, skip, msg


def kernel(x1, edge_index1, edge_attr1, batch1, x2, edge_index2, edge_attr2, batch2, t_value, params):
    pad_e = EP - E
    batch1c = batch1.reshape(N, 1).astype(jnp.int32)
    batch2c = batch2.reshape(N, 1).astype(jnp.int32)
    x2c = x2.reshape(N, 1).astype(jnp.int32)

    ea1 = jnp.pad(edge_attr1[:, 0:1], ((0, pad_e), (0, 0)))
    ea2 = jnp.pad(edge_attr2[:, 0:1], ((0, pad_e), (0, 0)))
    s1 = jnp.pad(edge_index1[0], (0, pad_e)).astype(jnp.int32).reshape(ER, 128)
    d1 = jnp.pad(edge_index1[1], (0, pad_e)).astype(jnp.int32).reshape(ER, 128)
    s2 = jnp.pad(edge_index2[0], (0, pad_e)).astype(jnp.int32).reshape(ER, 128)
    d2 = jnp.pad(edge_index2[1], (0, pad_e)).astype(jnp.int32).reshape(ER, 128)
    zeros_n = jnp.zeros((N,), _f32)

    t_enc, node_enc = _nodeenc(params, x2c, batch1c, t_value)
    nf = jnp.concatenate([t_enc, node_enc], axis=1)

    ee1 = _mlp3(params["edge_full"], ea1, ERB)
    ee1_init = ee1
    ee2 = _mlp3(params["edge_partial"], ea2, ERB)
    ee2_init = ee2

    for i in range(NB_LAYER):
        sc1, sk1, m1 = _tconv(params["gnn_g"][i], nf, d1, s1, ee1, zeros_n)
        o1 = _gnorm(params["gn_f"][i], sc1, sk1, batch1c)
        d2b = lax.optimization_barrier((d2, sc1))[0]
        sc2, sk2, m2 = _tconv(params["gnn_f"][i], nf, d2b, s2, ee2, zeros_n)
        o2 = _gnorm(params["gn_p"][i], sc2, sk2, batch2c)
        ee1 = m1
        ee2 = _mlp3res(params["inter"][i], m2, ee2_init, ERB)
        nf = jnp.concatenate([o1, o2], axis=1)

    out = _dec(params["dec"], ee1, ee1_init)
    return out[:E]
